# Initial kernel scaffold; baseline (speedup 1.0000x reference)
import jax
import jax.numpy as jnp
from jax.experimental import pallas as pl


def kernel(w1, b1, w2, b2, wfc1, bfc1, wfc2, bfc2, x):
    raise NotImplementedError("write your pallas kernel here")


# speedup vs baseline: 1.7250x; 1.7250x over previous
import jax
import jax.numpy as jnp
from jax import lax
from jax.experimental import pallas as pl
from jax.experimental.pallas import tpu as pltpu

_LIN = 8000
_XROWS = 528
_MP = 520
_S2 = 256
_F = 32768


def _conv_fused_kernel(xr_ref, w1e_ref, b1e_ref, w2p_ref, b2p_ref, o_ref):
    xr = xr_ref[...]
    p1 = jnp.concatenate([xr[u:u + _MP, :] for u in range(8)], axis=1)
    a1 = jnp.dot(p1, w1e_ref[...], preferred_element_type=jnp.float32)
    a1 = jnp.maximum(a1 + b1e_ref[...], 0.0)
    row = lax.broadcasted_iota(jnp.int32, (_MP, 128), 0)
    r = lax.broadcasted_iota(jnp.int32, (_MP, 128), 1) % 4
    pos = 4 * row + r - 10
    xd2t = jnp.where((pos >= 0) & (pos < 2000), a1, 0.0)
    v = xd2t.reshape(_MP // 2, 256)
    p2 = jnp.concatenate(
        [v[d // 2:d // 2 + _S2, 128 * (d % 2):128 * (d % 2) + 128]
         for d in range(8)], axis=1)
    a2 = jnp.dot(p2, w2p_ref[...], preferred_element_type=jnp.float32)
    a2 = jnp.maximum(a2 + b2p_ref[...], 0.0)
    o_ref[...] = a2


def _fc_kernel(h_ref, w1_ref, b1_ref, w2_ref, b2_ref, o_ref):
    z = lax.dot_general(h_ref[...], w1_ref[...], (((1,), (1,)), ((), ())),
                        preferred_element_type=jnp.float32)
    z = jnp.maximum(z + b1_ref[...], 0.0)
    logits = lax.dot_general(z, w2_ref[...], (((1,), (1,)), ((), ())),
                             preferred_element_type=jnp.float32)
    o_ref[...] = logits + b2_ref[...]


def kernel(w1, b1, w2, b2, wfc1, bfc1, wfc2, bfc2, x):
    B = x.shape[0]
    xp = jnp.pad(x[:, 0, :], ((0, 0), (88, _XROWS * 16 - 88 - _LIN)))
    xr = xp.reshape(B, _XROWS, 16)

    w1f = w1[:, 0, :]
    w1e = jnp.stack([jnp.pad(w1f, ((0, 0), (4 * r, 12 - 4 * r)))
                     for r in range(4)])
    w1e = w1e.transpose(2, 1, 0).reshape(112, 128)
    w1e = jnp.pad(w1e, ((0, 16), (0, 0)))
    b1e = jnp.repeat(b1, 4)[None, :]

    d_i = jnp.arange(8)[:, None, None]
    jl_i = jnp.arange(2)[None, :, None]
    r_i = jnp.arange(4)[None, None, :]
    k_idx = 4 * (d_i - jl_i) + r_i
    valid = (k_idx >= 0) & (k_idx < 25)
    g = w2[:, :, jnp.clip(k_idx, 0, 24)]
    g = g * valid[None, None].astype(w2.dtype)
    w2p = g.transpose(2, 1, 4, 3, 0).reshape(1024, 128)
    b2p = jnp.tile(b2, 2)[None, :]

    H = wfc1.shape[0]
    w1p = wfc1.reshape(H, 64, 500).transpose(0, 2, 1)
    w1p = jnp.pad(w1p, ((0, 0), (0, 12), (0, 0))).reshape(H, _F)

    h2 = pl.pallas_call(
        _conv_fused_kernel,
        out_shape=jax.ShapeDtypeStruct((B, _S2, 128), jnp.float32),
        grid=(B,),
        in_specs=[
            pl.BlockSpec((None, _XROWS, 16), lambda b: (b, 0, 0)),
            pl.BlockSpec((128, 128), lambda b: (0, 0)),
            pl.BlockSpec((1, 128), lambda b: (0, 0)),
            pl.BlockSpec((1024, 128), lambda b: (0, 0)),
            pl.BlockSpec((1, 128), lambda b: (0, 0)),
        ],
        out_specs=pl.BlockSpec((None, _S2, 128), lambda b: (b, 0, 0)),
        compiler_params=pltpu.CompilerParams(
            dimension_semantics=("parallel",)),
    )(xr, w1e, b1e, w2p, b2p)

    hf = h2.reshape(B, _F)
    C = wfc2.shape[0]
    return pl.pallas_call(
        _fc_kernel,
        out_shape=jax.ShapeDtypeStruct((B, C), jnp.float32),
        grid=(2,),
        in_specs=[
            pl.BlockSpec((B // 2, _F), lambda i: (i, 0)),
            pl.BlockSpec((H, _F), lambda i: (0, 0)),
            pl.BlockSpec((1, H), lambda i: (0, 0)),
            pl.BlockSpec((C, H), lambda i: (0, 0)),
            pl.BlockSpec((1, C), lambda i: (0, 0)),
        ],
        out_specs=pl.BlockSpec((B // 2, C), lambda i: (i, 0)),
        compiler_params=pltpu.CompilerParams(
            dimension_semantics=("parallel",)),
    )(hf, w1p, bfc1.reshape(1, H), wfc2, bfc2.reshape(1, C))
